# Initial kernel scaffold; baseline (speedup 1.0000x reference)
import jax
import jax.numpy as jnp
from jax.experimental import pallas as pl


def kernel(x, edge_index, W1, b1, W2, b2, a1w, a1b, a2w, a2b):
    raise NotImplementedError("write your pallas kernel here")


# speedup vs baseline: 13.4560x; 13.4560x over previous
import functools

import jax
import jax.numpy as jnp
from jax import lax
from jax.experimental import pallas as pl
from jax.experimental.pallas import tpu as pltpu
from jax.experimental.pallas import tpu_sc as plsc

N = 10000
E = 320000
D = 128
NP = 10240
NC = 2
NS = 16
RPT = NP // (NC * NS) * NC
EPS = E // NC
EPT = EPS // NS
K = 80
NB = EPT // K
B = 1024

_mesh = plsc.VectorSubcoreMesh(core_axis_name="c", subcore_axis_name="s")


@functools.partial(
    pl.kernel,
    out_type=jax.ShapeDtypeStruct((NC, NP), jnp.float32),
    mesh=_mesh,
    scratch_types=[
        pltpu.VMEM((K,), jnp.int32),
        pltpu.VMEM((K,), jnp.float32),
        pltpu.VMEM((RPT,), jnp.float32),
        pltpu.VMEM_SHARED((NP,), jnp.float32),
    ],
)
def _sc_degree(dst_hbm, out_hbm, dst_v, ones_v, stage_v, deg_sh):
    c = lax.axis_index("c")
    s = lax.axis_index("s")
    for j in range(K // 16):
        ones_v[pl.ds(j * 16, 16)] = jnp.ones((16,), jnp.float32)

    def zb(r, carry):
        stage_v[pl.ds(r * 16, 16)] = jnp.zeros((16,), jnp.float32)
        return carry

    lax.fori_loop(0, RPT // 16, zb, 0)
    r0 = s * RPT
    pltpu.sync_copy(stage_v, deg_sh.at[pl.ds(r0, RPT)])
    plsc.subcore_barrier()

    base = c * EPS + s * EPT

    def body(b_i, carry):
        off = base + b_i * K
        pltpu.sync_copy(dst_hbm.at[pl.ds(off, K)], dst_v)
        pltpu.sync_copy(ones_v, deg_sh.at[dst_v], add=True)
        return carry

    lax.fori_loop(0, NB, body, 0)
    plsc.subcore_barrier()
    pltpu.sync_copy(deg_sh.at[pl.ds(r0, RPT)], stage_v)
    pltpu.sync_copy(stage_v, out_hbm.at[c, pl.ds(r0, RPT)])


@functools.partial(
    pl.kernel,
    out_type=jax.ShapeDtypeStruct((NC, NP, D), jnp.float32),
    mesh=_mesh,
    scratch_types=[
        pltpu.VMEM((K,), jnp.int32),
        pltpu.VMEM((K,), jnp.int32),
        pltpu.VMEM((K, D), jnp.float32),
        pltpu.VMEM_SHARED((NP, D), jnp.float32),
        pltpu.SemaphoreType.DMA,
    ],
)
def _sc_edge_agg(h_hbm, src_hbm, dst_hbm, out_hbm, src_v, dst_v, rows_v,
                 agg_sh, sem):
    c = lax.axis_index("c")
    s = lax.axis_index("s")

    def zb(r, carry):
        for j in range(D // 16):
            rows_v[r, pl.ds(j * 16, 16)] = jnp.zeros((16,), jnp.float32)
        return carry

    lax.fori_loop(0, K, zb, 0)
    r0 = s * RPT
    for j in range(RPT // K):
        pltpu.sync_copy(rows_v, agg_sh.at[pl.ds(r0 + j * K, K)])
    plsc.subcore_barrier()

    base = c * EPS + s * EPT

    def body(b_i, carry):
        off = base + b_i * K
        pltpu.sync_copy(src_hbm.at[pl.ds(off, K)], src_v)
        pltpu.sync_copy(dst_hbm.at[pl.ds(off, K)], dst_v)
        pltpu.async_copy(h_hbm.at[src_v], rows_v, sem).wait()
        pltpu.sync_copy(rows_v, agg_sh.at[dst_v], add=True)
        return carry

    lax.fori_loop(0, NB, body, 0)
    plsc.subcore_barrier()
    for j in range(RPT // K):
        pltpu.sync_copy(agg_sh.at[pl.ds(r0 + j * K, K)], rows_v)
        pltpu.sync_copy(rows_v, out_hbm.at[c, pl.ds(r0 + j * K, K)])


def _tc1_body(x_ref, w_ref, degp_ref, h1p_ref, dinv_ref):
    d = degp_ref[0] + degp_ref[1] + 1.0
    dinv = lax.rsqrt(jnp.maximum(d, 1e-12))
    dinv_ref[...] = dinv
    h1p_ref[...] = dinv * jnp.dot(x_ref[...], w_ref[...],
                                  preferred_element_type=jnp.float32)


def _tc1(xp, W1, degp3):
    return pl.pallas_call(
        _tc1_body,
        grid=(NP // B,),
        in_specs=[
            pl.BlockSpec((B, D), lambda i: (i, 0)),
            pl.BlockSpec((D, D), lambda i: (0, 0)),
            pl.BlockSpec((NC, B, 1), lambda i: (0, i, 0)),
        ],
        out_specs=[
            pl.BlockSpec((B, D), lambda i: (i, 0)),
            pl.BlockSpec((B, 1), lambda i: (i, 0)),
        ],
        out_shape=[
            jax.ShapeDtypeStruct((NP, D), jnp.float32),
            jax.ShapeDtypeStruct((NP, 1), jnp.float32),
        ],
    )(xp, W1, degp3)


def _tc2_body(aggp_ref, hp_ref, dinv_ref, b_ref, aw_ref, ab_ref, w2_ref,
              out_ref):
    dinv = dinv_ref[...]
    t = dinv * (aggp_ref[0] + aggp_ref[1] + hp_ref[...]) + b_ref[...]
    t = jnp.maximum(t, 0.0)
    s = jax.nn.sigmoid(jnp.dot(t, aw_ref[...],
                               preferred_element_type=jnp.float32) + ab_ref[...])
    out_ref[...] = dinv * jnp.dot(t * s, w2_ref[...],
                                  preferred_element_type=jnp.float32)


def _tc2(aggp, hp, dinv, b1r, a1w, a1br, W2):
    return pl.pallas_call(
        _tc2_body,
        grid=(NP // B,),
        in_specs=[
            pl.BlockSpec((NC, B, D), lambda i: (0, i, 0)),
            pl.BlockSpec((B, D), lambda i: (i, 0)),
            pl.BlockSpec((B, 1), lambda i: (i, 0)),
            pl.BlockSpec((1, D), lambda i: (0, 0)),
            pl.BlockSpec((D, 1), lambda i: (0, 0)),
            pl.BlockSpec((1, 1), lambda i: (0, 0)),
            pl.BlockSpec((D, D), lambda i: (0, 0)),
        ],
        out_specs=pl.BlockSpec((B, D), lambda i: (i, 0)),
        out_shape=jax.ShapeDtypeStruct((NP, D), jnp.float32),
    )(aggp, hp, dinv, b1r, a1w, a1br, W2)


def _tc3_body(aggp_ref, hp_ref, dinv_ref, b_ref, aw_ref, ab_ref, out_ref):
    t = dinv_ref[...] * (aggp_ref[0] + aggp_ref[1] + hp_ref[...]) + b_ref[...]
    s = jax.nn.sigmoid(jnp.dot(t, aw_ref[...],
                               preferred_element_type=jnp.float32) + ab_ref[...])
    out_ref[...] = t * s


def _tc3(aggp, hp, dinv, b2r, a2w, a2br):
    return pl.pallas_call(
        _tc3_body,
        grid=(NP // B,),
        in_specs=[
            pl.BlockSpec((NC, B, D), lambda i: (0, i, 0)),
            pl.BlockSpec((B, D), lambda i: (i, 0)),
            pl.BlockSpec((B, 1), lambda i: (i, 0)),
            pl.BlockSpec((1, D), lambda i: (0, 0)),
            pl.BlockSpec((D, 1), lambda i: (0, 0)),
            pl.BlockSpec((1, 1), lambda i: (0, 0)),
        ],
        out_specs=pl.BlockSpec((B, D), lambda i: (i, 0)),
        out_shape=jax.ShapeDtypeStruct((NP, D), jnp.float32),
    )(aggp, hp, dinv, b2r, a2w, a2br)


def kernel(x, edge_index, W1, b1, W2, b2, a1w, a1b, a2w, a2b):
    src = edge_index[0]
    dst = edge_index[1]
    xp = jnp.pad(x, ((0, NP - N), (0, 0)))

    degp = _sc_degree(dst)
    degp3 = degp.reshape(NC, NP, 1)
    h1p, dinv = _tc1(xp, W1, degp3)
    agg1 = _sc_edge_agg(h1p, src, dst)
    h2p = _tc2(agg1, h1p, dinv, b1.reshape(1, D), a1w, a1b.reshape(1, 1), W2)
    agg2 = _sc_edge_agg(h2p, src, dst)
    out = _tc3(agg2, h2p, dinv, b2.reshape(1, D), a2w, a2b.reshape(1, 1))
    return out[:N]
